# baseline (speedup 1.0000x reference)
import jax
import jax.numpy as jnp
from jax import lax
from jax.experimental import pallas as pl
from jax.experimental.pallas import tpu as pltpu
from jax.experimental.pallas import tpu_sc as plsc

_N_POINTS = 16384
_N_IN = 100000

_NC = 2
_NS = 16
_NW = _NC * _NS
_ROWS_PER_W = _N_POINTS // _NW
_G = 16


def _body(coords_hbm, idx_hbm, out_c, idx_v, stg, sem_s, sem_w):
    wid = lax.axis_index("s") * _NC + lax.axis_index("c")
    base = wid * _ROWS_PER_W
    pltpu.sync_copy(idx_hbm.at[pl.ds(base, _ROWS_PER_W)], idx_v)

    def group_body(g, carry):
        s = g * _G
        rows = idx_v[pl.ds(s, _G)]
        for lane in range(_G):
            r = rows[lane]
            pltpu.async_copy(coords_hbm.at[pl.ds(r, 1)],
                             stg.at[pl.ds(s + lane, 1)], sem_s)
        return carry

    lax.fori_loop(0, _ROWS_PER_W // _G, group_body, 0)
    pltpu.make_async_copy(coords_hbm.at[pl.ds(0, _ROWS_PER_W)], stg,
                          sem_s).wait()
    pltpu.sync_copy(stg, out_c.at[pl.ds(base, _ROWS_PER_W)])


@jax.jit
def _coords_gather(coords, idx32):
    run = pl.kernel(
        _body,
        out_type=jax.ShapeDtypeStruct((_N_POINTS, 3), jnp.float32),
        mesh=plsc.VectorSubcoreMesh(core_axis_name="c", subcore_axis_name="s"),
        scratch_types=[
            pltpu.VMEM((_ROWS_PER_W,), jnp.int32),
            pltpu.VMEM((_ROWS_PER_W, 3), jnp.float32),
            pltpu.SemaphoreType.DMA,
            pltpu.SemaphoreType.DMA,
        ],
    )
    return run(coords, idx32)


def kernel(coords, features, colors, normals, idx):
    idx32 = idx.astype(jnp.int32)
    out_c = _coords_gather(coords, idx32)
    out_f = jnp.take(features, idx32, axis=0)
    out_col = jnp.take(colors, idx32, axis=0)
    out_n = jnp.take(normals, idx32, axis=0)
    return (out_c, out_f, out_col, out_n)


# speedup vs baseline: 1.1425x; 1.1425x over previous
import jax
import jax.numpy as jnp
from jax import lax
from jax.experimental import pallas as pl
from jax.experimental.pallas import tpu as pltpu
from jax.experimental.pallas import tpu_sc as plsc

_N_POINTS = 16384
_N_IN = 100000
_D_FEAT = 128

_NC = 2
_NS = 16
_NW = _NC * _NS
_ROWS_PER_W = _N_POINTS // _NW
_FC = 128
_NFC = _ROWS_PER_W // _FC
_G = 16


def _body(coords_hbm, features_hbm, colors_hbm, normals_hbm, idx_hbm,
          out_c, out_f, out_col, out_n,
          idx_v, fa, fb, raw,
          sem_f0, sem_f1, sem_w0, sem_w1, sem_s, sem_sw):
    wid = lax.axis_index("s") * _NC + lax.axis_index("c")
    base = wid * _ROWS_PER_W

    pltpu.sync_copy(idx_hbm.at[pl.ds(base, _ROWS_PER_W)], idx_v)

    fbufs = (fa, fb)
    fsems = (sem_f0, sem_f1)
    wsems = (sem_w0, sem_w1)

    def fgather(c):
        return pltpu.async_copy(
            features_hbm.at[idx_v.at[pl.ds(c * _FC, _FC)]],
            fbufs[c % 2], fsems[c % 2])

    def fwrite(c):
        return pltpu.async_copy(
            fbufs[c % 2], out_f.at[pl.ds(base + c * _FC, _FC)], wsems[c % 2])

    def issue_rows(tbl):
        def group_body(g, carry):
            s = g * _G
            rows = idx_v[pl.ds(s, _G)]
            for lane in range(_G):
                pltpu.async_copy(tbl.at[pl.ds(rows[lane], 1)],
                                 raw.at[pl.ds(s + lane, 1)], sem_s)
            return carry
        lax.fori_loop(0, _ROWS_PER_W // _G, group_body, 0)

    def drain_rows(tbl):
        pltpu.make_async_copy(tbl.at[pl.ds(0, _ROWS_PER_W)], raw,
                              sem_s).wait()

    g0 = fgather(0)
    g1 = fgather(1)

    issue_rows(coords_hbm)
    g0.wait()
    w0 = fwrite(0)
    drain_rows(coords_hbm)
    sw0 = pltpu.async_copy(raw, out_c.at[pl.ds(base, _ROWS_PER_W)], sem_sw)
    g1.wait()
    w1 = fwrite(1)
    sw0.wait()

    issue_rows(colors_hbm)
    w0.wait()
    g2 = fgather(2)
    drain_rows(colors_hbm)
    sw1 = pltpu.async_copy(raw, out_col.at[pl.ds(base, _ROWS_PER_W)], sem_sw)
    w1.wait()
    g3 = fgather(3)
    sw1.wait()

    issue_rows(normals_hbm)
    g2.wait()
    w2 = fwrite(2)
    drain_rows(normals_hbm)
    sw2 = pltpu.async_copy(raw, out_n.at[pl.ds(base, _ROWS_PER_W)], sem_sw)
    g3.wait()
    w3 = fwrite(3)

    w2.wait()
    w3.wait()
    sw2.wait()


@jax.jit
def _downsample(coords, features, colors, normals, idx32):
    f32 = jnp.float32
    run = pl.kernel(
        _body,
        out_type=(
            jax.ShapeDtypeStruct((_N_POINTS, 3), f32),
            jax.ShapeDtypeStruct((_N_POINTS, _D_FEAT), f32),
            jax.ShapeDtypeStruct((_N_POINTS, 3), f32),
            jax.ShapeDtypeStruct((_N_POINTS, 3), f32),
        ),
        mesh=plsc.VectorSubcoreMesh(core_axis_name="c", subcore_axis_name="s"),
        scratch_types=[
            pltpu.VMEM((_ROWS_PER_W,), jnp.int32),
            pltpu.VMEM((_FC, _D_FEAT), f32),
            pltpu.VMEM((_FC, _D_FEAT), f32),
            pltpu.VMEM((_ROWS_PER_W, 3), f32),
            pltpu.SemaphoreType.DMA,
            pltpu.SemaphoreType.DMA,
            pltpu.SemaphoreType.DMA,
            pltpu.SemaphoreType.DMA,
            pltpu.SemaphoreType.DMA,
            pltpu.SemaphoreType.DMA,
        ],
    )
    return run(coords, features, colors, normals, idx32)


def kernel(coords, features, colors, normals, idx):
    idx32 = idx.astype(jnp.int32)
    out_c, out_f, out_col, out_n = _downsample(coords, features, colors,
                                               normals, idx32)
    return (out_c, out_f, out_col, out_n)
